# baseline (speedup 1.0000x reference)
import functools

import numpy as np
import jax
import jax.numpy as jnp
from jax import lax
from jax.experimental import pallas as pl
from jax.experimental.pallas import tpu as pltpu
from jax.experimental.pallas import tpu_sc as plsc

_VOCAB = 100000
_SEQ = 2048
_D = 512
_B = 4
_NC = 2
_NS = 16
_NW = _NC * _NS
_PW = _SEQ // _NW
_VPR = _D // 16
_CR = 32
_NCH = _B * _PW // _CR
_NBUF = 4


def _pos_encoding():
    i = np.arange(_D // 2, dtype=np.float64)
    denom = np.power(10000.0, 2.0 * i / _D)
    pos = np.arange(_SEQ, dtype=np.float64)[:, None]
    pe = np.zeros((_SEQ, _D), dtype=np.float64)
    pe[:, 0::2] = np.sin(pos / denom)
    pe[:, 1::2] = np.cos(pos / denom)
    return jnp.asarray(pe, dtype=jnp.float32)


_mesh = plsc.VectorSubcoreMesh(core_axis_name="c", subcore_axis_name="s")


@functools.partial(
    pl.kernel,
    mesh=_mesh,
    out_type=jax.ShapeDtypeStruct((_B * _SEQ, _D), jnp.float32),
    scratch_types=[
        pltpu.VMEM((_NCH, _CR), jnp.int32),
        pltpu.VMEM((_PW, _D), jnp.float32),
    ]
    + [pltpu.VMEM((_CR, _D), jnp.float32) for _ in range(_NBUF)]
    + [pltpu.SemaphoreType.DMA for _ in range(2 * _NBUF)],
)
def _emb_kernel(idx_hbm, table_hbm, pos_hbm, out_hbm, idx_v, pos_v, *bufs):
    rv = bufs[:_NBUF]
    gs = bufs[_NBUF:2 * _NBUF]
    osem = bufs[2 * _NBUF:]

    c = lax.axis_index("c")
    s = lax.axis_index("s")
    w = s * _NC + c
    p0 = w * _PW

    pltpu.sync_copy(idx_hbm.at[w], idx_v)
    pltpu.sync_copy(pos_hbm.at[pl.ds(p0, _PW)], pos_v)

    def gather(k):
        return pltpu.async_copy(
            table_hbm.at[idx_v.at[k]], rv[k % _NBUF], gs[k % _NBUF])

    def out_copy(k):
        row0 = (k // 2) * _SEQ + p0 + (k % 2) * _CR
        return pltpu.async_copy(
            rv[k % _NBUF], out_hbm.at[pl.ds(row0, _CR)], osem[k % _NBUF])

    def add_pos(k):
        row_ref = rv[k % _NBUF]
        r0 = (k % 2) * _CR

        def body(r):
            for j in range(_VPR):
                v = pos_v[r0 + r, pl.ds(j * 16, 16)]
                plsc.addupdate(row_ref.at[r, pl.ds(j * 16, 16)], v)

        plsc.parallel_loop(0, _CR, unroll=4)(body)

    gd = [None] * _NCH
    od = [None] * _NCH
    for k in range(_NBUF - 1):
        gd[k] = gather(k)
    for k in range(_NCH):
        gd[k].wait()
        if k + _NBUF - 1 < _NCH:
            if k >= 1:
                od[k - 1].wait()
            gd[k + _NBUF - 1] = gather(k + _NBUF - 1)
        add_pos(k)
        od[k] = out_copy(k)
    for k in range(_NCH - _NBUF, _NCH):
        od[k].wait()


def kernel(inputs, table):
    idx = (inputs.astype(jnp.int32)
           .reshape(_B, _NW, 2, _CR)
           .transpose(1, 0, 2, 3)
           .reshape(_NW, _NCH, _CR))
    out = _emb_kernel(idx, table, _pos_encoding())
    return out.reshape(_B, _SEQ, _D)


# speedup vs baseline: 1.0560x; 1.0560x over previous
import functools

import numpy as np
import jax
import jax.numpy as jnp
from jax import lax
from jax.experimental import pallas as pl
from jax.experimental.pallas import tpu as pltpu
from jax.experimental.pallas import tpu_sc as plsc

_VOCAB = 100000
_SEQ = 2048
_D = 512
_B = 4
_NC = 2
_NS = 16
_NW = _NC * _NS
_PW = _SEQ // _NW
_VPR = _D // 16
_HR = _PW // 2


def _pos_encoding():
    i = np.arange(_D // 2, dtype=np.float64)
    denom = np.power(10000.0, 2.0 * i / _D)
    pos = np.arange(_SEQ, dtype=np.float64)[:, None]
    pe = np.zeros((_SEQ, _D), dtype=np.float64)
    pe[:, 0::2] = np.sin(pos / denom)
    pe[:, 1::2] = np.cos(pos / denom)
    return jnp.asarray(pe, dtype=jnp.float32)


_mesh = plsc.VectorSubcoreMesh(core_axis_name="c", subcore_axis_name="s")


@functools.partial(
    pl.kernel,
    mesh=_mesh,
    out_type=jax.ShapeDtypeStruct((_B * _SEQ, _D), jnp.float32),
    scratch_types=[
        pltpu.VMEM((_B, _PW), jnp.int32),
        pltpu.VMEM((_PW, _D), jnp.float32),
        pltpu.VMEM((_PW, _D), jnp.float32),
        pltpu.VMEM((_PW, _D), jnp.float32),
        pltpu.SemaphoreType.DMA,
        pltpu.SemaphoreType.DMA,
        pltpu.SemaphoreType.DMA,
        pltpu.SemaphoreType.DMA,
        pltpu.SemaphoreType.DMA,
    ],
)
def _emb_kernel(idx_hbm, table_hbm, pos_hbm, out_hbm,
                idx_v, pos_v, rv0, rv1, gs0, gs1, os0, os1, ps):
    c = lax.axis_index("c")
    s = lax.axis_index("s")
    w = s * _NC + c
    p0 = w * _PW

    rv = (rv0, rv1)
    gs = (gs0, gs1)
    osem = (os0, os1)

    pd = pltpu.async_copy(pos_hbm.at[pl.ds(p0, _PW)], pos_v, ps)
    for b in range(_B):
        pltpu.sync_copy(idx_hbm.at[b, pl.ds(p0, _PW)], idx_v.at[b])

    def gather(b):
        return pltpu.async_copy(table_hbm.at[idx_v.at[b]], rv[b % 2], gs[b % 2])

    def out_half(b, h):
        row0 = b * _SEQ + p0 + h * _HR
        return pltpu.async_copy(
            rv[b % 2].at[pl.ds(h * _HR, _HR)],
            out_hbm.at[pl.ds(row0, _HR)], osem[b % 2])

    def add_half(b, h):
        row_ref = rv[b % 2]

        def body(r):
            for j in range(_VPR):
                v = pos_v[r, pl.ds(j * 16, 16)]
                plsc.addupdate(row_ref.at[r, pl.ds(j * 16, 16)], v)

        plsc.parallel_loop(h * _HR, (h + 1) * _HR, unroll=2)(body)

    gd = [None] * _B
    od = [[None, None] for _ in range(_B)]
    gd[0] = gather(0)
    gd[1] = gather(1)
    pd.wait()
    for b in range(_B):
        gd[b].wait()
        add_half(b, 0)
        od[b][0] = out_half(b, 0)
        add_half(b, 1)
        od[b][1] = out_half(b, 1)
        if b + 2 < _B:
            od[b][0].wait()
            od[b][1].wait()
            gd[b + 2] = gather(b + 2)
    od[_B - 2][0].wait()
    od[_B - 2][1].wait()
    od[_B - 1][0].wait()
    od[_B - 1][1].wait()


def kernel(inputs, table):
    out = _emb_kernel(inputs.astype(jnp.int32), table, _pos_encoding())
    return out.reshape(_B, _SEQ, _D)
